# baseline (device time: 598178 ns/iter reference)
import jax
import jax.numpy as jnp
from jax import lax
from jax.experimental import pallas as pl
from jax.experimental.pallas import tpu as pltpu

T = 1024
SND_ROWS = 576
CH = 32
M32 = SND_ROWS // CH
M8 = 3


def _align8(v):
    return (v // 8) * 8


def _exchange(g, c0_arr):
    d = g.shape[1]

    def body(c0_ref, g_ref, out_ref, send32, recv32, send8, recv8):
        my_x = lax.axis_index("x")
        my_y = lax.axis_index("y")
        my_z = lax.axis_index("z")
        partner = (my_x, 1 - my_y, my_z)
        y0 = my_y == 0

        c0v = c0_ref[0]
        n_send = jnp.where(y0, T - c0v, c0v)
        recv_start = jnp.where(y0, c0v, 0)
        r_dst = jnp.where(y0, 0, (T - c0v) % 8)
        a_dst = jnp.where(y0, 0, _align8(T - c0v))
        l8_out = _align8(r_dst + n_send + 7)
        m32_out = l8_out // CH
        rem8_out = (l8_out % CH) // 8
        l8_in = _align8(recv_start % 8 + n_send + 7)
        m32_in = l8_in // CH
        rem8_in = (l8_in % CH) // 8

        out_ref[:, :] = g_ref[pl.ds(0, T), :]

        barrier = pltpu.get_barrier_semaphore()
        pl.semaphore_signal(
            barrier, inc=1, device_id=partner,
            device_id_type=pl.DeviceIdType.MESH,
        )
        pl.semaphore_wait(barrier, 1)

        for k in range(M32):
            @pl.when(k < m32_out)
            def _(k=k):
                do = pl.multiple_of(a_dst + k * CH, 8)
                pltpu.make_async_remote_copy(
                    src_ref=g_ref.at[pl.ds(T + k * CH, CH), :],
                    dst_ref=out_ref.at[pl.ds(do, CH), :],
                    send_sem=send32.at[k],
                    recv_sem=recv32.at[k],
                    device_id=partner,
                    device_id_type=pl.DeviceIdType.MESH,
                ).start()
        for j in range(M8):
            @pl.when(j < rem8_out)
            def _(j=j):
                off = m32_out * CH + j * 8
                so = pl.multiple_of(T + off, 8)
                do = pl.multiple_of(a_dst + off, 8)
                pltpu.make_async_remote_copy(
                    src_ref=g_ref.at[pl.ds(so, 8), :],
                    dst_ref=out_ref.at[pl.ds(do, 8), :],
                    send_sem=send8.at[j],
                    recv_sem=recv8.at[j],
                    device_id=partner,
                    device_id_type=pl.DeviceIdType.MESH,
                ).start()

        def desc32(k):
            return pltpu.make_async_remote_copy(
                src_ref=g_ref.at[pl.ds(T, CH), :],
                dst_ref=out_ref.at[pl.ds(0, CH), :],
                send_sem=send32.at[k], recv_sem=recv32.at[k],
                device_id=partner, device_id_type=pl.DeviceIdType.MESH,
            )

        def desc8(j):
            return pltpu.make_async_remote_copy(
                src_ref=g_ref.at[pl.ds(T, 8), :],
                dst_ref=out_ref.at[pl.ds(0, 8), :],
                send_sem=send8.at[j], recv_sem=recv8.at[j],
                device_id=partner, device_id_type=pl.DeviceIdType.MESH,
            )

        for k in range(M32):
            @pl.when(k < m32_in)
            def _(k=k):
                desc32(k).wait_recv()
        for j in range(M8):
            @pl.when(j < rem8_in)
            def _(j=j):
                desc8(j).wait_recv()
        for k in range(M32):
            @pl.when(k < m32_out)
            def _(k=k):
                desc32(k).wait_send()
        for j in range(M8):
            @pl.when(j < rem8_out)
            def _(j=j):
                desc8(j).wait_send()

        iota = lax.broadcasted_iota(jnp.int32, (8, d), 0)

        a_h = pl.multiple_of(_align8(recv_start), 8)
        r_h = recv_start - a_h
        out_ref[pl.ds(a_h, 8), :] = jnp.where(
            iota < r_h, g_ref[pl.ds(a_h, 8), :], out_ref[pl.ds(a_h, 8), :])

        e = recv_start + n_send
        a_t = pl.multiple_of(jnp.minimum(_align8(e), T - 8), 8)
        r_t = e - a_t
        out_ref[pl.ds(a_t, 8), :] = jnp.where(
            iota >= r_t, g_ref[pl.ds(a_t, 8), :], out_ref[pl.ds(a_t, 8), :])

    return pl.pallas_call(
        body,
        out_shape=jax.ShapeDtypeStruct((T, d), g.dtype),
        in_specs=[
            pl.BlockSpec(memory_space=pltpu.SMEM),
            pl.BlockSpec(memory_space=pltpu.VMEM),
        ],
        out_specs=pl.BlockSpec(memory_space=pltpu.VMEM),
        scratch_shapes=[
            pltpu.SemaphoreType.DMA((M32,)),
            pltpu.SemaphoreType.DMA((M32,)),
            pltpu.SemaphoreType.DMA((M8,)),
            pltpu.SemaphoreType.DMA((M8,)),
        ],
        compiler_params=pltpu.CompilerParams(collective_id=0),
    )(c0_arr, g)


def kernel(x, dest):
    t, d = x.shape
    my_y = lax.axis_index("y")
    y0 = my_y == 0

    i = jnp.arange(t, dtype=jnp.int32)
    zeros = (dest == 0).astype(jnp.int32)
    c = jnp.cumsum(zeros)
    c0 = c[-1]
    cum1 = (i + 1) - c
    r_dst = jnp.where(y0, 0, (t - c0) % 8)

    keep_hay = jnp.where(y0, c, cum1)
    k_needle = jnp.where(y0, i, jnp.clip(i - c0, 0, t - 1)) + 1
    k_pos = jnp.searchsorted(keep_hay, k_needle, side="left")
    keep_mask = jnp.where(y0, i < c0, i >= c0)
    k_idx = jnp.where(keep_mask, k_pos, 0)

    j = jnp.arange(SND_ROWS, dtype=jnp.int32)
    send_hay = jnp.where(y0, cum1, c)
    s_needle = jnp.where(y0, j, jnp.clip(j - r_dst, 0, t - 1)) + 1
    s_pos = jnp.searchsorted(send_hay, s_needle, side="left")
    s_idx = jnp.where(y0 | (j >= r_dst), s_pos, 0)

    idx = jnp.clip(jnp.concatenate([k_idx, s_idx]), 0, t - 1)
    g = jnp.take(x, idx, axis=0)

    return _exchange(g, jnp.reshape(c0, (1,)))


# device time: 482967 ns/iter; 1.2385x vs baseline; 1.2385x over previous
import jax
import jax.numpy as jnp
from jax import lax
from jax.experimental import pallas as pl
from jax.experimental.pallas import tpu as pltpu

T = 1024
SND_ROWS = 576
CH = 32
M32 = SND_ROWS // CH
M8 = 3


def _align8(v):
    return (v // 8) * 8


def _exchange(g, c0_arr):
    d = g.shape[1]

    def body(c0_ref, g_ref, out_ref, send32, recv32, send8, recv8):
        my_x = lax.axis_index("x")
        my_y = lax.axis_index("y")
        my_z = lax.axis_index("z")
        partner = (my_x, 1 - my_y, my_z)
        y0 = my_y == 0

        c0v = c0_ref[0]
        n_send = jnp.where(y0, T - c0v, c0v)
        recv_start = jnp.where(y0, c0v, 0)
        r_dst = jnp.where(y0, 0, (T - c0v) % 8)
        a_dst = jnp.where(y0, 0, _align8(T - c0v))
        l8_out = _align8(r_dst + n_send + 7)
        m32_out = l8_out // CH
        rem8_out = (l8_out % CH) // 8
        l8_in = _align8(recv_start % 8 + n_send + 7)
        m32_in = l8_in // CH
        rem8_in = (l8_in % CH) // 8

        out_ref[:, :] = g_ref[pl.ds(0, T), :]

        barrier = pltpu.get_barrier_semaphore()
        pl.semaphore_signal(
            barrier, inc=1, device_id=partner,
            device_id_type=pl.DeviceIdType.MESH,
        )
        pl.semaphore_wait(barrier, 1)

        for k in range(M32):
            @pl.when(k < m32_out)
            def _(k=k):
                do = pl.multiple_of(a_dst + k * CH, 8)
                pltpu.make_async_remote_copy(
                    src_ref=g_ref.at[pl.ds(T + k * CH, CH), :],
                    dst_ref=out_ref.at[pl.ds(do, CH), :],
                    send_sem=send32.at[k],
                    recv_sem=recv32.at[k],
                    device_id=partner,
                    device_id_type=pl.DeviceIdType.MESH,
                ).start()
        for j in range(M8):
            @pl.when(j < rem8_out)
            def _(j=j):
                off = m32_out * CH + j * 8
                so = pl.multiple_of(T + off, 8)
                do = pl.multiple_of(a_dst + off, 8)
                pltpu.make_async_remote_copy(
                    src_ref=g_ref.at[pl.ds(so, 8), :],
                    dst_ref=out_ref.at[pl.ds(do, 8), :],
                    send_sem=send8.at[j],
                    recv_sem=recv8.at[j],
                    device_id=partner,
                    device_id_type=pl.DeviceIdType.MESH,
                ).start()

        def desc32(k):
            return pltpu.make_async_remote_copy(
                src_ref=g_ref.at[pl.ds(T, CH), :],
                dst_ref=out_ref.at[pl.ds(0, CH), :],
                send_sem=send32.at[k], recv_sem=recv32.at[k],
                device_id=partner, device_id_type=pl.DeviceIdType.MESH,
            )

        def desc8(j):
            return pltpu.make_async_remote_copy(
                src_ref=g_ref.at[pl.ds(T, 8), :],
                dst_ref=out_ref.at[pl.ds(0, 8), :],
                send_sem=send8.at[j], recv_sem=recv8.at[j],
                device_id=partner, device_id_type=pl.DeviceIdType.MESH,
            )

        for k in range(M32):
            @pl.when(k < m32_in)
            def _(k=k):
                desc32(k).wait_recv()
        for j in range(M8):
            @pl.when(j < rem8_in)
            def _(j=j):
                desc8(j).wait_recv()
        for k in range(M32):
            @pl.when(k < m32_out)
            def _(k=k):
                desc32(k).wait_send()
        for j in range(M8):
            @pl.when(j < rem8_out)
            def _(j=j):
                desc8(j).wait_send()

        iota = lax.broadcasted_iota(jnp.int32, (8, d), 0)

        a_h = pl.multiple_of(_align8(recv_start), 8)
        r_h = recv_start - a_h
        out_ref[pl.ds(a_h, 8), :] = jnp.where(
            iota < r_h, g_ref[pl.ds(a_h, 8), :], out_ref[pl.ds(a_h, 8), :])

        e = recv_start + n_send
        a_t = pl.multiple_of(jnp.minimum(_align8(e), T - 8), 8)
        r_t = e - a_t
        out_ref[pl.ds(a_t, 8), :] = jnp.where(
            iota >= r_t, g_ref[pl.ds(a_t, 8), :], out_ref[pl.ds(a_t, 8), :])

    return pl.pallas_call(
        body,
        out_shape=jax.ShapeDtypeStruct((T, d), g.dtype),
        in_specs=[
            pl.BlockSpec(memory_space=pltpu.SMEM),
            pl.BlockSpec(memory_space=pltpu.VMEM),
        ],
        out_specs=pl.BlockSpec(memory_space=pltpu.VMEM),
        scratch_shapes=[
            pltpu.SemaphoreType.DMA((M32,)),
            pltpu.SemaphoreType.DMA((M32,)),
            pltpu.SemaphoreType.DMA((M8,)),
            pltpu.SemaphoreType.DMA((M8,)),
        ],
        compiler_params=pltpu.CompilerParams(collective_id=0),
    )(c0_arr, g)


def kernel(x, dest):
    t, d = x.shape
    my_y = lax.axis_index("y")
    y0 = my_y == 0

    i = jnp.arange(t, dtype=jnp.int32)
    zeros = (dest == 0).astype(jnp.int32)
    c = jnp.cumsum(zeros)
    c0 = c[-1]
    cum1 = (i + 1) - c
    r_dst = jnp.where(y0, 0, (t - c0) % 8)

    def ssorted(hay, needle):
        return jnp.sum(
            hay[None, :] < needle[:, None], axis=1, dtype=jnp.int32)

    keep_hay = jnp.where(y0, c, cum1)
    k_needle = jnp.where(y0, i, jnp.clip(i - c0, 0, t - 1)) + 1
    k_pos = ssorted(keep_hay, k_needle)
    keep_mask = jnp.where(y0, i < c0, i >= c0)
    k_idx = jnp.where(keep_mask, k_pos, 0)

    j = jnp.arange(SND_ROWS, dtype=jnp.int32)
    send_hay = jnp.where(y0, cum1, c)
    s_needle = jnp.where(y0, j, jnp.clip(j - r_dst, 0, t - 1)) + 1
    s_pos = ssorted(send_hay, s_needle)
    s_idx = jnp.where(y0 | (j >= r_dst), s_pos, 0)

    idx = jnp.clip(jnp.concatenate([k_idx, s_idx]), 0, t - 1)
    g = jnp.take(x, idx, axis=0)

    return _exchange(g, jnp.reshape(c0, (1,)))


# device time: 19125 ns/iter; 31.2773x vs baseline; 25.2532x over previous
import jax
import jax.numpy as jnp
from jax import lax
from jax.experimental import pallas as pl
from jax.experimental.pallas import tpu as pltpu

T = 1024
SND_ROWS = 576
CH = 32
M32 = SND_ROWS // CH
M8 = 3


def _align8(v):
    return (v // 8) * 8


def _exchange(x, k_idx, s_idx, c0_arr):
    d = x.shape[1]

    def body(c0_ref, kidx_ref, sidx_ref, x_ref, out_ref,
             snd_scr, rcv_scr, send32, recv32, send8, recv8):
        my_x = lax.axis_index("x")
        my_y = lax.axis_index("y")
        my_z = lax.axis_index("z")
        partner = (my_x, 1 - my_y, my_z)
        y0 = my_y == 0

        c0v = c0_ref[0]
        n_send = jnp.where(y0, T - c0v, c0v)
        recv_start = jnp.where(y0, c0v, 0)
        r_dst = jnp.where(y0, 0, (T - c0v) % 8)
        l8_out = _align8(r_dst + n_send + 7)
        m32_out = l8_out // CH
        rem8_out = (l8_out % CH) // 8
        r_fix = recv_start % 8
        l8_in = _align8(r_fix + n_send + 7)
        m32_in = l8_in // CH
        rem8_in = (l8_in % CH) // 8

        xb = x_ref[:, :].astype(jnp.bfloat16)

        oh_s = (lax.broadcasted_iota(jnp.int32, (SND_ROWS, T), 1)
                == sidx_ref[:, :]).astype(jnp.bfloat16)
        snd_scr[:, :] = jax.lax.dot_general(
            oh_s, xb, (((1,), (0,)), ((), ())),
            preferred_element_type=jnp.float32).astype(jnp.bfloat16)

        barrier = pltpu.get_barrier_semaphore()
        pl.semaphore_signal(
            barrier, inc=1, device_id=partner,
            device_id_type=pl.DeviceIdType.MESH,
        )
        pl.semaphore_wait(barrier, 1)

        for k in range(M32):
            @pl.when(k < m32_out)
            def _(k=k):
                pltpu.make_async_remote_copy(
                    src_ref=snd_scr.at[pl.ds(k * CH, CH), :],
                    dst_ref=rcv_scr.at[pl.ds(k * CH, CH), :],
                    send_sem=send32.at[k],
                    recv_sem=recv32.at[k],
                    device_id=partner,
                    device_id_type=pl.DeviceIdType.MESH,
                ).start()
        for j in range(M8):
            @pl.when(j < rem8_out)
            def _(j=j):
                off = pl.multiple_of(m32_out * CH + j * 8, 8)
                pltpu.make_async_remote_copy(
                    src_ref=snd_scr.at[pl.ds(off, 8), :],
                    dst_ref=rcv_scr.at[pl.ds(off, 8), :],
                    send_sem=send8.at[j],
                    recv_sem=recv8.at[j],
                    device_id=partner,
                    device_id_type=pl.DeviceIdType.MESH,
                ).start()

        oh_k = (lax.broadcasted_iota(jnp.int32, (T, T), 1)
                == kidx_ref[:, :]).astype(jnp.bfloat16)
        out_ref[:, :] = jax.lax.dot_general(
            oh_k, xb, (((1,), (0,)), ((), ())),
            preferred_element_type=jnp.float32)

        def desc32(k):
            return pltpu.make_async_remote_copy(
                src_ref=snd_scr.at[pl.ds(0, CH), :],
                dst_ref=rcv_scr.at[pl.ds(0, CH), :],
                send_sem=send32.at[k], recv_sem=recv32.at[k],
                device_id=partner, device_id_type=pl.DeviceIdType.MESH,
            )

        def desc8(j):
            return pltpu.make_async_remote_copy(
                src_ref=snd_scr.at[pl.ds(0, 8), :],
                dst_ref=rcv_scr.at[pl.ds(0, 8), :],
                send_sem=send8.at[j], recv_sem=recv8.at[j],
                device_id=partner, device_id_type=pl.DeviceIdType.MESH,
            )

        for k in range(M32):
            @pl.when(k < m32_in)
            def _(k=k):
                desc32(k).wait_recv()
        for j in range(M8):
            @pl.when(j < rem8_in)
            def _(j=j):
                desc8(j).wait_recv()

        a_h = pl.multiple_of(_align8(recv_start), 8)

        def blend(off, rows, it):
            q = it + off
            mask = (q >= r_fix) & (q < r_fix + n_send)
            dst = pl.ds(pl.multiple_of(a_h + off, 8), rows)
            out_ref[dst, :] = jnp.where(
                mask,
                rcv_scr[pl.ds(off, rows), :].astype(jnp.float32),
                out_ref[dst, :])

        it32 = lax.broadcasted_iota(jnp.int32, (CH, d), 0)
        it8 = lax.broadcasted_iota(jnp.int32, (8, d), 0)
        for k in range(M32):
            @pl.when(k < m32_in)
            def _(k=k):
                blend(k * CH, CH, it32)
        for j in range(M8):
            @pl.when(j < rem8_in)
            def _(j=j):
                blend(m32_in * CH + j * 8, 8, it8)

        for k in range(M32):
            @pl.when(k < m32_out)
            def _(k=k):
                desc32(k).wait_send()
        for j in range(M8):
            @pl.when(j < rem8_out)
            def _(j=j):
                desc8(j).wait_send()

    return pl.pallas_call(
        body,
        out_shape=jax.ShapeDtypeStruct((T, d), x.dtype),
        in_specs=[
            pl.BlockSpec(memory_space=pltpu.SMEM),
            pl.BlockSpec(memory_space=pltpu.VMEM),
            pl.BlockSpec(memory_space=pltpu.VMEM),
            pl.BlockSpec(memory_space=pltpu.VMEM),
        ],
        out_specs=pl.BlockSpec(memory_space=pltpu.VMEM),
        scratch_shapes=[
            pltpu.VMEM((SND_ROWS, d), jnp.bfloat16),
            pltpu.VMEM((SND_ROWS, d), jnp.bfloat16),
            pltpu.SemaphoreType.DMA((M32,)),
            pltpu.SemaphoreType.DMA((M32,)),
            pltpu.SemaphoreType.DMA((M8,)),
            pltpu.SemaphoreType.DMA((M8,)),
        ],
        compiler_params=pltpu.CompilerParams(collective_id=0),
    )(c0_arr, k_idx, s_idx, x)


def kernel(x, dest):
    t, d = x.shape
    my_y = lax.axis_index("y")
    y0 = my_y == 0

    i = jnp.arange(t, dtype=jnp.int32)
    zeros = (dest == 0).astype(jnp.int32)
    c = jnp.cumsum(zeros)
    c0 = c[-1]
    cum1 = (i + 1) - c
    r_dst = jnp.where(y0, 0, (t - c0) % 8)

    def ssorted(hay, needle):
        return jnp.sum(
            hay[None, :] < needle[:, None], axis=1, dtype=jnp.int32)

    keep_hay = jnp.where(y0, c, cum1)
    k_needle = jnp.where(y0, i, jnp.clip(i - c0, 0, t - 1)) + 1
    k_pos = ssorted(keep_hay, k_needle)
    keep_mask = jnp.where(y0, i < c0, i >= c0)
    k_idx = jnp.where(keep_mask, k_pos, 0)

    j = jnp.arange(SND_ROWS, dtype=jnp.int32)
    send_hay = jnp.where(y0, cum1, c)
    s_needle = jnp.where(y0, j, jnp.clip(j - r_dst, 0, t - 1)) + 1
    s_pos = ssorted(send_hay, s_needle)
    s_idx = jnp.where(y0 | (j >= r_dst), s_pos, 0)

    k_idx = jnp.clip(k_idx, 0, t - 1).reshape(t, 1)
    s_idx = jnp.clip(s_idx, 0, t - 1).reshape(SND_ROWS, 1)
    return _exchange(x, k_idx, s_idx, jnp.reshape(c0, (1,)))


# device time: 16934 ns/iter; 35.3241x vs baseline; 1.1294x over previous
import jax
import jax.numpy as jnp
from jax import lax
from jax.experimental import pallas as pl
from jax.experimental.pallas import tpu as pltpu

T = 1024
SND_ROWS = 576
CH = 32
M32 = SND_ROWS // CH
M8 = 3


def _align8(v):
    return (v // 8) * 8


def _exchange(x, k_idx, s_idx, c0_arr):
    d = x.shape[1]

    def body(c0_ref, kidx_ref, sidx_ref, x_ref, out_ref,
             snd_scr, rcv_scr, send32, recv32, send8, recv8):
        my_x = lax.axis_index("x")
        my_y = lax.axis_index("y")
        my_z = lax.axis_index("z")
        partner = (my_x, 1 - my_y, my_z)
        y0 = my_y == 0

        c0v = c0_ref[0]
        n_send = jnp.where(y0, T - c0v, c0v)
        recv_start = jnp.where(y0, c0v, 0)
        r_dst = jnp.where(y0, 0, (T - c0v) % 8)
        l8_out = _align8(r_dst + n_send + 7)
        m32_out = l8_out // CH
        rem8_out = (l8_out % CH) // 8
        r_fix = recv_start % 8
        l8_in = _align8(r_fix + n_send + 7)
        m32_in = l8_in // CH
        rem8_in = (l8_in % CH) // 8

        xb = x_ref[:, :].astype(jnp.bfloat16)

        oh_s = (lax.broadcasted_iota(jnp.int32, (SND_ROWS, T), 0)
                == sidx_ref[:, :]).astype(jnp.bfloat16)
        snd_scr[:, :] = jax.lax.dot_general(
            oh_s, xb, (((1,), (0,)), ((), ())),
            preferred_element_type=jnp.float32).astype(jnp.bfloat16)

        barrier = pltpu.get_barrier_semaphore()
        pl.semaphore_signal(
            barrier, inc=1, device_id=partner,
            device_id_type=pl.DeviceIdType.MESH,
        )
        pl.semaphore_wait(barrier, 1)

        for k in range(M32):
            @pl.when(k < m32_out)
            def _(k=k):
                pltpu.make_async_remote_copy(
                    src_ref=snd_scr.at[pl.ds(k * CH, CH), :],
                    dst_ref=rcv_scr.at[pl.ds(k * CH, CH), :],
                    send_sem=send32.at[k],
                    recv_sem=recv32.at[k],
                    device_id=partner,
                    device_id_type=pl.DeviceIdType.MESH,
                ).start()
        for j in range(M8):
            @pl.when(j < rem8_out)
            def _(j=j):
                off = pl.multiple_of(m32_out * CH + j * 8, 8)
                pltpu.make_async_remote_copy(
                    src_ref=snd_scr.at[pl.ds(off, 8), :],
                    dst_ref=rcv_scr.at[pl.ds(off, 8), :],
                    send_sem=send8.at[j],
                    recv_sem=recv8.at[j],
                    device_id=partner,
                    device_id_type=pl.DeviceIdType.MESH,
                ).start()

        oh_k = (lax.broadcasted_iota(jnp.int32, (T, T), 0)
                == kidx_ref[:, :]).astype(jnp.bfloat16)
        out_ref[:, :] = jax.lax.dot_general(
            oh_k, xb, (((1,), (0,)), ((), ())),
            preferred_element_type=jnp.float32)

        def desc32(k):
            return pltpu.make_async_remote_copy(
                src_ref=snd_scr.at[pl.ds(0, CH), :],
                dst_ref=rcv_scr.at[pl.ds(0, CH), :],
                send_sem=send32.at[k], recv_sem=recv32.at[k],
                device_id=partner, device_id_type=pl.DeviceIdType.MESH,
            )

        def desc8(j):
            return pltpu.make_async_remote_copy(
                src_ref=snd_scr.at[pl.ds(0, 8), :],
                dst_ref=rcv_scr.at[pl.ds(0, 8), :],
                send_sem=send8.at[j], recv_sem=recv8.at[j],
                device_id=partner, device_id_type=pl.DeviceIdType.MESH,
            )

        for k in range(M32):
            @pl.when(k < m32_in)
            def _(k=k):
                desc32(k).wait_recv()
        for j in range(M8):
            @pl.when(j < rem8_in)
            def _(j=j):
                desc8(j).wait_recv()

        a_h = pl.multiple_of(_align8(recv_start), 8)

        def blend(off, rows, it):
            q = it + off
            mask = (q >= r_fix) & (q < r_fix + n_send)
            dst = pl.ds(pl.multiple_of(a_h + off, 8), rows)
            out_ref[dst, :] = jnp.where(
                mask,
                rcv_scr[pl.ds(off, rows), :].astype(jnp.float32),
                out_ref[dst, :])

        it32 = lax.broadcasted_iota(jnp.int32, (CH, d), 0)
        it8 = lax.broadcasted_iota(jnp.int32, (8, d), 0)
        for k in range(M32):
            @pl.when(k < m32_in)
            def _(k=k):
                blend(k * CH, CH, it32)
        for j in range(M8):
            @pl.when(j < rem8_in)
            def _(j=j):
                blend(m32_in * CH + j * 8, 8, it8)

        for k in range(M32):
            @pl.when(k < m32_out)
            def _(k=k):
                desc32(k).wait_send()
        for j in range(M8):
            @pl.when(j < rem8_out)
            def _(j=j):
                desc8(j).wait_send()

    return pl.pallas_call(
        body,
        out_shape=jax.ShapeDtypeStruct((T, d), x.dtype),
        in_specs=[
            pl.BlockSpec(memory_space=pltpu.SMEM),
            pl.BlockSpec(memory_space=pltpu.VMEM),
            pl.BlockSpec(memory_space=pltpu.VMEM),
            pl.BlockSpec(memory_space=pltpu.VMEM),
        ],
        out_specs=pl.BlockSpec(memory_space=pltpu.VMEM),
        scratch_shapes=[
            pltpu.VMEM((SND_ROWS, d), jnp.bfloat16),
            pltpu.VMEM((SND_ROWS, d), jnp.bfloat16),
            pltpu.SemaphoreType.DMA((M32,)),
            pltpu.SemaphoreType.DMA((M32,)),
            pltpu.SemaphoreType.DMA((M8,)),
            pltpu.SemaphoreType.DMA((M8,)),
        ],
        compiler_params=pltpu.CompilerParams(collective_id=0),
    )(c0_arr, k_idx, s_idx, x)


def kernel(x, dest):
    t, d = x.shape
    my_y = lax.axis_index("y")
    y0 = my_y == 0

    i = jnp.arange(t, dtype=jnp.int32)
    zeros = (dest == 0).astype(jnp.int32)
    c = jnp.cumsum(zeros)
    c0 = c[-1]
    cum1 = (i + 1) - c
    r_dst = jnp.where(y0, 0, (t - c0) % 8)

    keep_mask = jnp.where(y0, zeros == 1, zeros == 0)
    fwd_k = jnp.where(keep_mask,
                      jnp.where(y0, c - 1, c0 + cum1 - 1), -1)
    fwd_s = jnp.where(keep_mask, -1,
                      jnp.where(y0, cum1 - 1, r_dst + c - 1))

    return _exchange(x, fwd_k.reshape(1, t), fwd_s.reshape(1, t),
                     jnp.reshape(c0, (1,)))
